# Initial kernel scaffold; baseline (speedup 1.0000x reference)
import jax
import jax.numpy as jnp
from jax.experimental import pallas as pl


def kernel(A, H0, w1, b1, w2, b2):
    raise NotImplementedError("write your pallas kernel here")


# speedup vs baseline: 1.2846x; 1.2846x over previous
import jax
import jax.numpy as jnp
from jax import lax
from jax.experimental import pallas as pl
from jax.experimental.pallas import tpu as pltpu

_VMEM_LIMIT = 40 * 1024 * 1024
_F32 = jnp.float32


def _y1_kernel(h0_ref, w1t_ref, y1_ref):
    y1_ref[...] = jnp.dot(h0_ref[...], w1t_ref[...],
                          preferred_element_type=_F32)


def _layer1_kernel(a_ref, y1_ref, b1_ref, w2t_ref, y2_ref):
    acc = jnp.dot(a_ref[...], y1_ref[...], preferred_element_type=_F32)
    h1 = jnp.maximum(acc + b1_ref[...], 0.0)
    y2_ref[...] = jnp.dot(h1, w2t_ref[...], preferred_element_type=_F32)


def _layer2_kernel(a_ref, y2_ref, b2_ref, h2_ref):
    acc = jnp.dot(a_ref[...], y2_ref[...], preferred_element_type=_F32)
    h2_ref[...] = acc + b2_ref[...]


def _decoder_kernel(h2i_ref, h2all_ref, recon_ref):
    logits = lax.dot_general(
        h2i_ref[...], h2all_ref[...],
        dimension_numbers=(((1,), (1,)), ((), ())),
        preferred_element_type=_F32)
    recon_ref[...] = 0.5 * jnp.tanh(0.5 * logits) + 0.5


def kernel(A, H0, w1, b1, w2, b2):
    N = A.shape[0]
    d0 = H0.shape[1]
    d1 = w1.shape[0]
    d2 = w2.shape[0]

    A = A.astype(_F32)
    H0 = H0.astype(_F32)
    W1t = w1.astype(_F32).T
    W2t = w2.astype(_F32).T
    b1 = jnp.reshape(b1, (1, d1)).astype(_F32)
    b2 = jnp.reshape(b2, (1, d2)).astype(_F32)

    tm = 256 if N % 256 == 0 else (128 if N % 128 == 0 else N)
    n_row = N // tm

    par = pltpu.CompilerParams(dimension_semantics=("parallel",),
                               vmem_limit_bytes=_VMEM_LIMIT)

    y1 = pl.pallas_call(
        _y1_kernel,
        out_shape=jax.ShapeDtypeStruct((N, d1), _F32),
        grid=(n_row,),
        in_specs=[pl.BlockSpec((tm, d0), lambda i: (i, 0)),
                  pl.BlockSpec((d0, d1), lambda i: (0, 0))],
        out_specs=pl.BlockSpec((tm, d1), lambda i: (i, 0)),
        compiler_params=par,
    )(H0, W1t)

    y2 = pl.pallas_call(
        _layer1_kernel,
        out_shape=jax.ShapeDtypeStruct((N, d2), _F32),
        grid=(n_row,),
        in_specs=[pl.BlockSpec((tm, N), lambda i: (i, 0)),
                  pl.BlockSpec((N, d1), lambda i: (0, 0)),
                  pl.BlockSpec((1, d1), lambda i: (0, 0)),
                  pl.BlockSpec((d1, d2), lambda i: (0, 0))],
        out_specs=pl.BlockSpec((tm, d2), lambda i: (i, 0)),
        compiler_params=par,
    )(A, y1, b1, W2t)

    h2 = pl.pallas_call(
        _layer2_kernel,
        out_shape=jax.ShapeDtypeStruct((N, d2), _F32),
        grid=(n_row,),
        in_specs=[pl.BlockSpec((tm, N), lambda i: (i, 0)),
                  pl.BlockSpec((N, d2), lambda i: (0, 0)),
                  pl.BlockSpec((1, d2), lambda i: (0, 0))],
        out_specs=pl.BlockSpec((tm, d2), lambda i: (i, 0)),
        compiler_params=par,
    )(A, y2, b2)

    recon = pl.pallas_call(
        _decoder_kernel,
        out_shape=jax.ShapeDtypeStruct((N, N), _F32),
        grid=(n_row,),
        in_specs=[pl.BlockSpec((tm, d2), lambda i: (i, 0)),
                  pl.BlockSpec((N, d2), lambda i: (0, 0))],
        out_specs=pl.BlockSpec((tm, N), lambda i: (i, 0)),
        compiler_params=par,
    )(h2, h2)

    return recon, h2
